# Initial kernel scaffold; baseline (speedup 1.0000x reference)
import jax
import jax.numpy as jnp
from jax.experimental import pallas as pl


def kernel(h_nodes, h_edges, edit_i, edit_j, edit_b, feasible, stop_feasible, W_edit, b_edit, W_stop, b_stop):
    raise NotImplementedError("write your pallas kernel here")


# speedup vs baseline: 13.7113x; 13.7113x over previous
import functools

import jax
import jax.numpy as jnp
from jax import lax
from jax.experimental import pallas as pl
from jax.experimental.pallas import tpu as pltpu
from jax.experimental.pallas import tpu_sc as plsc

N_NODES = 512
NODE_DIM = 256
EDGE_DIM = 128
N_EDITS = 200000
N_BOND = 4

_NC = 2
_NS = 16
_NW = _NC * _NS
_LANES = 128
_P_EDITS = 200704
_ROWS = _P_EDITS // _LANES
_RPW = _ROWS // _NW


def _proj_body(h_ref, w12_ref, wstop_ref, bias8_ref, bstop_ref, sfeas_ref,
               ab_ref, stop_ref):
    h = h_ref[...]
    ab = jnp.dot(h, w12_ref[...], preferred_element_type=jnp.float32)
    ab_ref[...] = ab + bias8_ref[...]
    mean = jnp.mean(h, axis=0, keepdims=True)
    s = jnp.dot(mean, wstop_ref[...], preferred_element_type=jnp.float32)
    s = s + bstop_ref[...]
    stop_ref[...] = jnp.where(sfeas_ref[0, 0] > 0, s, -jnp.inf)


def _table_body(he_ref, a_ref, b_ref, w3_ref, t_ref):
    blk = he_ref.shape[0]
    he = he_ref[...].reshape(blk * N_NODES, EDGE_DIM)
    e = jnp.dot(he, w3_ref[...], preferred_element_type=jnp.float32)
    e = e.reshape(blk, N_NODES, N_BOND)
    t_ref[...] = e + a_ref[...][:, None, :] + b_ref[...][None, :, :]


def _sc_body(t_hbm, ei_hbm, ej_hbm, eb_hbm, ef_hbm, out_hbm,
             ei_v, ej_v, eb_v, ef_v, idx_v, val_v, sem):
    cid = lax.axis_index("c")
    sid = lax.axis_index("s")
    wid = sid * _NC + cid

    pltpu.sync_copy(ei_hbm.at[wid], ei_v)
    pltpu.sync_copy(ej_hbm.at[wid], ej_v)
    pltpu.sync_copy(eb_hbm.at[wid], eb_v)
    pltpu.sync_copy(ef_hbm.at[wid], ef_v)

    def idx_row(r, carry):
        for c in range(_LANES // 16):
            sl = pl.ds(c * 16, 16)
            i16 = ei_v[r, sl]
            j16 = ej_v[r, sl]
            b16 = eb_v[r, sl]
            idx_v[r, sl] = i16 * (N_NODES * N_BOND) + j16 * N_BOND + b16
        return carry

    lax.fori_loop(0, _RPW, idx_row, 0, unroll=False)

    def fire(r, carry):
        pltpu.make_async_copy(t_hbm.at[idx_v.at[r]], val_v.at[r], sem).start()
        return carry

    lax.fori_loop(0, _RPW, fire, 0, unroll=False)

    def drain(r, carry):
        pltpu.make_async_copy(t_hbm.at[idx_v.at[r]], val_v.at[r], sem).wait()
        return carry

    lax.fori_loop(0, _RPW, drain, 0, unroll=False)

    ninf = jnp.full((16,), -jnp.inf, dtype=jnp.float32)

    def mask_row(r, carry):
        for c in range(_LANES // 16):
            sl = pl.ds(c * 16, 16)
            val_v[r, sl] = jnp.where(ef_v[r, sl] > 0, val_v[r, sl], ninf)
        return carry

    lax.fori_loop(0, _RPW, mask_row, 0, unroll=False)

    pltpu.sync_copy(val_v, out_hbm.at[wid])


def _make_sc_gather():
    return pl.kernel(
        _sc_body,
        out_type=jax.ShapeDtypeStruct((_NW, _RPW, _LANES), jnp.float32),
        mesh=plsc.VectorSubcoreMesh(core_axis_name="c", subcore_axis_name="s",
                                    num_cores=_NC, num_subcores=_NS),
        scratch_types=[
            pltpu.VMEM((_RPW, _LANES), jnp.int32),
            pltpu.VMEM((_RPW, _LANES), jnp.int32),
            pltpu.VMEM((_RPW, _LANES), jnp.int32),
            pltpu.VMEM((_RPW, _LANES), jnp.int32),
            pltpu.VMEM((_RPW, _LANES), jnp.int32),
            pltpu.VMEM((_RPW, _LANES), jnp.float32),
            pltpu.SemaphoreType.DMA,
        ],
    )

_I_BLK = 8


def kernel(h_nodes, h_edges, edit_i, edit_j, edit_b, feasible, stop_feasible,
           W_edit, b_edit, W_stop, b_stop):
    w12 = jnp.concatenate([W_edit[:NODE_DIM], W_edit[NODE_DIM:2 * NODE_DIM]],
                          axis=1)
    w3 = W_edit[2 * NODE_DIM:]
    bias8 = jnp.concatenate([b_edit, jnp.zeros_like(b_edit)]).reshape(1, 8)
    sfeas = jnp.asarray(stop_feasible, jnp.int32).reshape(1, 1)

    ab, stop = pl.pallas_call(
        _proj_body,
        out_shape=(
            jax.ShapeDtypeStruct((N_NODES, 2 * N_BOND), jnp.float32),
            jax.ShapeDtypeStruct((1, 1), jnp.float32),
        ),
    )(h_nodes, w12, W_stop, bias8, b_stop.reshape(1, 1), sfeas)

    table = pl.pallas_call(
        _table_body,
        grid=(N_NODES // _I_BLK,),
        in_specs=[
            pl.BlockSpec((_I_BLK, N_NODES, EDGE_DIM), lambda i: (i, 0, 0)),
            pl.BlockSpec((_I_BLK, N_BOND), lambda i: (i, 0)),
            pl.BlockSpec((N_NODES, N_BOND), lambda i: (0, 0)),
            pl.BlockSpec((EDGE_DIM, N_BOND), lambda i: (0, 0)),
        ],
        out_specs=pl.BlockSpec((_I_BLK, N_NODES, N_BOND), lambda i: (i, 0, 0)),
        out_shape=jax.ShapeDtypeStruct((N_NODES, N_NODES, N_BOND), jnp.float32),
    )(h_edges, ab[:, :N_BOND], ab[:, N_BOND:], w3)

    pad = _P_EDITS - N_EDITS
    shp = (_NW, _RPW, _LANES)
    ei2 = jnp.pad(edit_i, (0, pad)).reshape(shp).astype(jnp.int32)
    ej2 = jnp.pad(edit_j, (0, pad)).reshape(shp).astype(jnp.int32)
    eb2 = jnp.pad(edit_b, (0, pad)).reshape(shp).astype(jnp.int32)
    ef2 = jnp.pad(feasible, (0, pad)).reshape(shp).astype(jnp.int32)

    out2 = _make_sc_gather()(table.reshape(-1), ei2, ej2, eb2, ef2)

    return jnp.concatenate([out2.reshape(-1)[:N_EDITS], stop.reshape(1)])


# speedup vs baseline: 25.1612x; 1.8351x over previous
import jax
import jax.numpy as jnp
from jax import lax
from jax.experimental import pallas as pl
from jax.experimental.pallas import tpu as pltpu
from jax.experimental.pallas import tpu_sc as plsc

N_NODES = 512
NODE_DIM = 256
EDGE_DIM = 128
N_EDITS = 200000
N_BOND = 4

_NC = 2
_NS = 16
_NW = _NC * _NS
_LANES = 128
_P_EDITS = 200704
_RPW = _P_EDITS // (_NW * _LANES)

_I_BLK = 8
_BLK_P = _I_BLK * N_NODES


def _proj_body(h_ref, w1p_ref, w2p_ref, wstop_ref, biasa_ref, bstop_ref,
               sfeas_ref, a_ref, bt_ref, stop_ref):
    h = h_ref[...]
    a_ref[...] = jnp.dot(h, w1p_ref[...],
                         preferred_element_type=jnp.float32) + biasa_ref[...]
    bt_ref[...] = lax.dot_general(w2p_ref[...], h, (((0,), (1,)), ((), ())),
                                  preferred_element_type=jnp.float32)
    mean = jnp.mean(h, axis=0, keepdims=True)
    s = jnp.dot(mean, wstop_ref[...], preferred_element_type=jnp.float32)
    s = s + bstop_ref[...]
    stop_ref[...] = jnp.where(sfeas_ref[0, 0] > 0, s, -jnp.inf)


def _table_body(he_ref, w3p_ref, a_ref, bt_ref, oh_ref, t_ref):
    he = he_ref[...].reshape(_BLK_P, EDGE_DIM)
    e = lax.dot_general(w3p_ref[...], he, (((0,), (1,)), ((), ())),
                        preferred_element_type=jnp.float32)
    ea = lax.dot_general(a_ref[...], oh_ref[...], (((0,), (1,)), ((), ())),
                         preferred_element_type=jnp.float32)
    bt = bt_ref[...]
    btile = jnp.concatenate([bt] * _I_BLK, axis=1)
    t_ref[...] = e + ea + btile


def _sc_body(t_hbm, ei_hbm, ej_hbm, eb_hbm, ef_hbm, out_hbm,
             ei_v, ej_v, eb_v, ef_v, idx_v, val_v, sem):
    cid = lax.axis_index("c")
    sid = lax.axis_index("s")
    wid = sid * _NC + cid

    pltpu.sync_copy(ei_hbm.at[wid], ei_v)
    pltpu.sync_copy(ej_hbm.at[wid], ej_v)
    pltpu.sync_copy(eb_hbm.at[wid], eb_v)
    pltpu.sync_copy(ef_hbm.at[wid], ef_v)

    def idx_row(r, carry):
        for c in range(_LANES // 16):
            sl = pl.ds(c * 16, 16)
            i16 = ei_v[r, sl]
            j16 = ej_v[r, sl]
            b16 = eb_v[r, sl]
            idx_v[r, sl] = b16 * (N_NODES * N_NODES) + i16 * N_NODES + j16
        return carry

    lax.fori_loop(0, _RPW, idx_row, 0, unroll=False)

    def fire(r, carry):
        pltpu.make_async_copy(t_hbm.at[idx_v.at[r]], val_v.at[r], sem).start()
        return carry

    lax.fori_loop(0, _RPW, fire, 0, unroll=False)

    def drain(r, carry):
        pltpu.make_async_copy(t_hbm.at[idx_v.at[r]], val_v.at[r], sem).wait()
        return carry

    lax.fori_loop(0, _RPW, drain, 0, unroll=False)

    ninf = jnp.full((16,), -jnp.inf, dtype=jnp.float32)

    def mask_row(r, carry):
        for c in range(_LANES // 16):
            sl = pl.ds(c * 16, 16)
            val_v[r, sl] = jnp.where(ef_v[r, sl] > 0, val_v[r, sl], ninf)
        return carry

    lax.fori_loop(0, _RPW, mask_row, 0, unroll=False)

    pltpu.sync_copy(val_v, out_hbm.at[wid])


def _make_sc_gather():
    return pl.kernel(
        _sc_body,
        out_type=jax.ShapeDtypeStruct((_NW, _RPW, _LANES), jnp.float32),
        mesh=plsc.VectorSubcoreMesh(core_axis_name="c", subcore_axis_name="s",
                                    num_cores=_NC, num_subcores=_NS),
        scratch_types=[
            pltpu.VMEM((_RPW, _LANES), jnp.int32),
            pltpu.VMEM((_RPW, _LANES), jnp.int32),
            pltpu.VMEM((_RPW, _LANES), jnp.int32),
            pltpu.VMEM((_RPW, _LANES), jnp.int32),
            pltpu.VMEM((_RPW, _LANES), jnp.int32),
            pltpu.VMEM((_RPW, _LANES), jnp.float32),
            pltpu.SemaphoreType.DMA,
        ],
    )


def kernel(h_nodes, h_edges, edit_i, edit_j, edit_b, feasible, stop_feasible,
           W_edit, b_edit, W_stop, b_stop):
    zpad = jnp.zeros((NODE_DIM, N_BOND), jnp.float32)
    w1p = jnp.concatenate([W_edit[:NODE_DIM], zpad], axis=1)
    w2p = jnp.concatenate([W_edit[NODE_DIM:2 * NODE_DIM], zpad], axis=1)
    w3p = jnp.pad(W_edit[2 * NODE_DIM:], ((0, 0), (0, 4)))
    biasa = jnp.concatenate([b_edit, jnp.zeros_like(b_edit)]).reshape(1, 8)
    sfeas = jnp.asarray(stop_feasible, jnp.int32).reshape(1, 1)
    onehot = (jnp.arange(_BLK_P, dtype=jnp.int32)[:, None] // N_NODES
              == jnp.arange(_I_BLK, dtype=jnp.int32)[None, :]
              ).astype(jnp.float32)

    a_full, bt, stop = pl.pallas_call(
        _proj_body,
        out_shape=(
            jax.ShapeDtypeStruct((N_NODES, 2 * N_BOND), jnp.float32),
            jax.ShapeDtypeStruct((2 * N_BOND, N_NODES), jnp.float32),
            jax.ShapeDtypeStruct((1, 1), jnp.float32),
        ),
    )(h_nodes, w1p, w2p, W_stop, biasa, b_stop.reshape(1, 1), sfeas)

    table = pl.pallas_call(
        _table_body,
        grid=(N_NODES // _I_BLK,),
        in_specs=[
            pl.BlockSpec((_I_BLK, N_NODES, EDGE_DIM), lambda i: (i, 0, 0)),
            pl.BlockSpec((EDGE_DIM, 2 * N_BOND), lambda i: (0, 0)),
            pl.BlockSpec((_I_BLK, 2 * N_BOND), lambda i: (i, 0)),
            pl.BlockSpec((2 * N_BOND, N_NODES), lambda i: (0, 0)),
            pl.BlockSpec((_BLK_P, _I_BLK), lambda i: (0, 0)),
        ],
        out_specs=pl.BlockSpec((2 * N_BOND, _BLK_P), lambda i: (0, i)),
        out_shape=jax.ShapeDtypeStruct((2 * N_BOND, N_NODES * N_NODES),
                                       jnp.float32),
    )(h_edges, w3p, a_full, bt, onehot)

    pad = _P_EDITS - N_EDITS
    shp = (_NW, _RPW, _LANES)
    ei2 = jnp.pad(edit_i, (0, pad)).reshape(shp).astype(jnp.int32)
    ej2 = jnp.pad(edit_j, (0, pad)).reshape(shp).astype(jnp.int32)
    eb2 = jnp.pad(edit_b, (0, pad)).reshape(shp).astype(jnp.int32)
    ef2 = jnp.pad(feasible, (0, pad)).reshape(shp).astype(jnp.int32)

    out2 = _make_sc_gather()(table.reshape(-1), ei2, ej2, eb2, ef2)

    return jnp.concatenate([out2.reshape(-1)[:N_EDITS], stop.reshape(1)])


# speedup vs baseline: 30.1860x; 1.1997x over previous
import jax
import jax.numpy as jnp
from jax import lax
from jax.experimental import pallas as pl
from jax.experimental.pallas import tpu as pltpu
from jax.experimental.pallas import tpu_sc as plsc

N_NODES = 512
NODE_DIM = 256
EDGE_DIM = 128
N_EDITS = 200000
N_BOND = 4

_NC = 2
_NS = 16
_NW = _NC * _NS
_EPW = 6272
_P_EDITS = _NW * _EPW
_ROWS = _EPW // 128
_CHUNKS = _EPW // 16

_I_BLK = 32
_BLK_P = _I_BLK * N_NODES
_NQ = _BLK_P // 128


def _proj_body(h_ref, w1p_ref, w2p_ref, wstop_ref, biasa_ref, bstop_ref,
               sfeas_ref, a_ref, bt_ref, stop_ref):
    h = h_ref[...]
    a_ref[...] = jnp.dot(h, w1p_ref[...],
                         preferred_element_type=jnp.float32) + biasa_ref[...]
    bt_ref[...] = lax.dot_general(w2p_ref[...], h, (((0,), (1,)), ((), ())),
                                  preferred_element_type=jnp.float32)
    mean = jnp.mean(h, axis=0, keepdims=True)
    s = jnp.dot(mean, wstop_ref[...], preferred_element_type=jnp.float32)
    s = s + bstop_ref[...]
    stop_ref[...] = jnp.where(sfeas_ref[0, 0] > 0, s, -jnp.inf)


def _table_body(he_ref, w3p_ref, a_ref, bt_ref, oh_ref, t_ref):
    he = he_ref[...].reshape(_BLK_P, EDGE_DIM)
    e = lax.dot_general(w3p_ref[...], he, (((0,), (1,)), ((), ())),
                        preferred_element_type=jnp.float32)
    ea = lax.dot_general(a_ref[...], oh_ref[...], (((0,), (1,)), ((), ())),
                         preferred_element_type=jnp.float32)
    btile = jnp.concatenate([bt_ref[...]] * _I_BLK, axis=1)
    t = e + ea + btile
    for q in range(_NQ):
        t_ref[q] = t[:, q * 128:(q + 1) * 128]


def _sc_body(t_hbm, ei_hbm, ej_hbm, eb_hbm, ef_hbm, out_hbm,
             ei_v, ej_v, eb_v, ef_v, idx_v, val_v, sem):
    cid = lax.axis_index("c")
    sid = lax.axis_index("s")
    wid = sid * _NC + cid
    base = wid * _EPW

    pltpu.sync_copy(ei_hbm.at[pl.ds(base, _EPW)], ei_v)
    pltpu.sync_copy(ej_hbm.at[pl.ds(base, _EPW)], ej_v)
    pltpu.sync_copy(eb_hbm.at[pl.ds(base, _EPW)], eb_v)
    pltpu.sync_copy(ef_hbm.at[pl.ds(base, _EPW)], ef_v)

    def idx_chunk(c, carry):
        sl = pl.ds(c * 16, 16)
        i16 = ei_v[sl]
        j16 = ej_v[sl]
        b16 = eb_v[sl]
        idx_v[sl] = ((i16 << 12) + ((j16 >> 7) << 10) + (b16 << 7)
                     + (j16 & 127))
        return carry

    lax.fori_loop(0, _CHUNKS, idx_chunk, 0, unroll=False)

    def fire(r, carry):
        sl = pl.ds(r * 128, 128)
        pltpu.make_async_copy(t_hbm.at[idx_v.at[sl]], val_v.at[sl],
                              sem).start()
        return carry

    lax.fori_loop(0, _ROWS, fire, 0, unroll=False)

    def drain(r, carry):
        sl = pl.ds(r * 128, 128)
        pltpu.make_async_copy(t_hbm.at[idx_v.at[sl]], val_v.at[sl], sem).wait()
        return carry

    lax.fori_loop(0, _ROWS, drain, 0, unroll=False)

    ninf = jnp.full((16,), -jnp.inf, dtype=jnp.float32)

    def mask_chunk(c, carry):
        sl = pl.ds(c * 16, 16)
        val_v[sl] = jnp.where(ef_v[sl] > 0, val_v[sl], ninf)
        return carry

    lax.fori_loop(0, _CHUNKS, mask_chunk, 0, unroll=False)

    pltpu.sync_copy(val_v, out_hbm.at[pl.ds(base, _EPW)])


def _make_sc_gather():
    return pl.kernel(
        _sc_body,
        out_type=jax.ShapeDtypeStruct((_P_EDITS,), jnp.float32),
        mesh=plsc.VectorSubcoreMesh(core_axis_name="c", subcore_axis_name="s",
                                    num_cores=_NC, num_subcores=_NS),
        scratch_types=[
            pltpu.VMEM((_EPW,), jnp.int32),
            pltpu.VMEM((_EPW,), jnp.int32),
            pltpu.VMEM((_EPW,), jnp.int32),
            pltpu.VMEM((_EPW,), jnp.int32),
            pltpu.VMEM((_EPW,), jnp.int32),
            pltpu.VMEM((_EPW,), jnp.float32),
            pltpu.SemaphoreType.DMA,
        ],
    )


def kernel(h_nodes, h_edges, edit_i, edit_j, edit_b, feasible, stop_feasible,
           W_edit, b_edit, W_stop, b_stop):
    zpad = jnp.zeros((NODE_DIM, N_BOND), jnp.float32)
    w1p = jnp.concatenate([W_edit[:NODE_DIM], zpad], axis=1)
    w2p = jnp.concatenate([W_edit[NODE_DIM:2 * NODE_DIM], zpad], axis=1)
    w3p = jnp.pad(W_edit[2 * NODE_DIM:], ((0, 0), (0, 4)))
    biasa = jnp.concatenate([b_edit, jnp.zeros_like(b_edit)]).reshape(1, 8)
    sfeas = jnp.asarray(stop_feasible, jnp.int32).reshape(1, 1)
    onehot = (jnp.arange(_BLK_P, dtype=jnp.int32)[:, None] // N_NODES
              == jnp.arange(_I_BLK, dtype=jnp.int32)[None, :]
              ).astype(jnp.float32)

    a_full, bt, stop = pl.pallas_call(
        _proj_body,
        out_shape=(
            jax.ShapeDtypeStruct((N_NODES, 2 * N_BOND), jnp.float32),
            jax.ShapeDtypeStruct((2 * N_BOND, N_NODES), jnp.float32),
            jax.ShapeDtypeStruct((1, 1), jnp.float32),
        ),
    )(h_nodes, w1p, w2p, W_stop, biasa, b_stop.reshape(1, 1), sfeas)

    table = pl.pallas_call(
        _table_body,
        grid=(N_NODES // _I_BLK,),
        in_specs=[
            pl.BlockSpec((_I_BLK, N_NODES, EDGE_DIM), lambda i: (i, 0, 0)),
            pl.BlockSpec((EDGE_DIM, 2 * N_BOND), lambda i: (0, 0)),
            pl.BlockSpec((_I_BLK, 2 * N_BOND), lambda i: (i, 0)),
            pl.BlockSpec((2 * N_BOND, N_NODES), lambda i: (0, 0)),
            pl.BlockSpec((_BLK_P, _I_BLK), lambda i: (0, 0)),
        ],
        out_specs=pl.BlockSpec((_NQ, 2 * N_BOND, 128), lambda i: (i, 0, 0)),
        out_shape=jax.ShapeDtypeStruct((N_NODES * N_NODES // 128,
                                        2 * N_BOND, 128), jnp.float32),
    )(h_edges, w3p, a_full, bt, onehot)

    pad = _P_EDITS - N_EDITS
    ei1 = jnp.pad(edit_i, (0, pad)).astype(jnp.int32)
    ej1 = jnp.pad(edit_j, (0, pad)).astype(jnp.int32)
    eb1 = jnp.pad(edit_b, (0, pad)).astype(jnp.int32)
    ef1 = jnp.pad(feasible, (0, pad)).astype(jnp.int32)

    out1 = _make_sc_gather()(table.reshape(-1), ei1, ej1, eb1, ef1)

    return jnp.concatenate([out1[:N_EDITS], stop.reshape(1)])


# speedup vs baseline: 30.9177x; 1.0242x over previous
import jax
import jax.numpy as jnp
from jax import lax
from jax.experimental import pallas as pl
from jax.experimental.pallas import tpu as pltpu
from jax.experimental.pallas import tpu_sc as plsc

N_NODES = 512
NODE_DIM = 256
EDGE_DIM = 128
N_EDITS = 200000
N_BOND = 4

_NC = 2
_NS = 16
_NW = _NC * _NS
_EPW = 6272
_P_EDITS = _NW * _EPW
_ROWS = _EPW // 128
_CHUNKS = _EPW // 16

_I_BLK = 32
_BLK_P = _I_BLK * N_NODES
_NQ = _BLK_P // 128


def _proj_body(h_ref, w1p_ref, w2p_ref, wstop_ref, biasa_ref, bstop_ref,
               sfeas_ref, a_ref, bt_ref, stop_ref):
    h = h_ref[...]
    a_ref[...] = jnp.dot(h, w1p_ref[...],
                         preferred_element_type=jnp.float32) + biasa_ref[...]
    bt_ref[...] = lax.dot_general(w2p_ref[...], h, (((0,), (1,)), ((), ())),
                                  preferred_element_type=jnp.float32)
    mean = jnp.mean(h, axis=0, keepdims=True)
    s = jnp.dot(mean, wstop_ref[...], preferred_element_type=jnp.float32)
    s = s + bstop_ref[...]
    stop_ref[...] = jnp.where(sfeas_ref[0, 0] > 0, s, -jnp.inf)


def _table_body(he_ref, w3p_ref, a_ref, bt_ref, oh_ref, t_ref):
    he = he_ref[...].reshape(_BLK_P, EDGE_DIM)
    e = lax.dot_general(w3p_ref[...], he, (((0,), (1,)), ((), ())),
                        preferred_element_type=jnp.float32)
    ea = lax.dot_general(a_ref[...], oh_ref[...], (((0,), (1,)), ((), ())),
                         preferred_element_type=jnp.float32)
    btile = jnp.concatenate([bt_ref[...]] * _I_BLK, axis=1)
    t = e + ea + btile
    for q in range(_NQ):
        t_ref[q] = t[:, q * 128:(q + 1) * 128]


def _sc_body(t_hbm, ei_hbm, ej_hbm, eb_hbm, ef_hbm, out_hbm,
             ei_v, ej_v, eb_v, ef_v, idx_v, val_v, sem):
    cid = lax.axis_index("c")
    sid = lax.axis_index("s")
    wid = sid * _NC + cid
    base = wid * _EPW

    pltpu.sync_copy(ei_hbm.at[pl.ds(base, _EPW)], ei_v)
    pltpu.sync_copy(ej_hbm.at[pl.ds(base, _EPW)], ej_v)
    pltpu.sync_copy(eb_hbm.at[pl.ds(base, _EPW)], eb_v)
    pltpu.sync_copy(ef_hbm.at[pl.ds(base, _EPW)], ef_v)

    def idx_fire_row(r, carry):
        for c in range(8):
            sl = pl.ds(r * 128 + c * 16, 16)
            i16 = ei_v[sl]
            j16 = ej_v[sl]
            b16 = eb_v[sl]
            idx_v[sl] = ((i16 << 12) + ((j16 >> 7) << 10) + (b16 << 7)
                         + (j16 & 127))
        sl = pl.ds(r * 128, 128)
        pltpu.make_async_copy(t_hbm.at[idx_v.at[sl]], val_v.at[sl],
                              sem).start()
        return carry

    lax.fori_loop(0, _ROWS, idx_fire_row, 0, unroll=False)

    ninf = jnp.full((16,), -jnp.inf, dtype=jnp.float32)

    def drain_mask_row(r, carry):
        sl = pl.ds(r * 128, 128)
        pltpu.make_async_copy(t_hbm.at[idx_v.at[sl]], val_v.at[sl], sem).wait()
        for c in range(8):
            slc = pl.ds(r * 128 + c * 16, 16)
            val_v[slc] = jnp.where(ef_v[slc] > 0, val_v[slc], ninf)
        return carry

    lax.fori_loop(0, _ROWS, drain_mask_row, 0, unroll=False)

    pltpu.sync_copy(val_v, out_hbm.at[pl.ds(base, _EPW)])


def _make_sc_gather():
    return pl.kernel(
        _sc_body,
        out_type=jax.ShapeDtypeStruct((_P_EDITS,), jnp.float32),
        mesh=plsc.VectorSubcoreMesh(core_axis_name="c", subcore_axis_name="s",
                                    num_cores=_NC, num_subcores=_NS),
        scratch_types=[
            pltpu.VMEM((_EPW,), jnp.int32),
            pltpu.VMEM((_EPW,), jnp.int32),
            pltpu.VMEM((_EPW,), jnp.int32),
            pltpu.VMEM((_EPW,), jnp.int32),
            pltpu.VMEM((_EPW,), jnp.int32),
            pltpu.VMEM((_EPW,), jnp.float32),
            pltpu.SemaphoreType.DMA,
        ],
    )


def kernel(h_nodes, h_edges, edit_i, edit_j, edit_b, feasible, stop_feasible,
           W_edit, b_edit, W_stop, b_stop):
    zpad = jnp.zeros((NODE_DIM, N_BOND), jnp.float32)
    w1p = jnp.concatenate([W_edit[:NODE_DIM], zpad], axis=1)
    w2p = jnp.concatenate([W_edit[NODE_DIM:2 * NODE_DIM], zpad], axis=1)
    w3p = jnp.pad(W_edit[2 * NODE_DIM:], ((0, 0), (0, 4)))
    biasa = jnp.concatenate([b_edit, jnp.zeros_like(b_edit)]).reshape(1, 8)
    sfeas = jnp.asarray(stop_feasible, jnp.int32).reshape(1, 1)
    onehot = (jnp.arange(_BLK_P, dtype=jnp.int32)[:, None] // N_NODES
              == jnp.arange(_I_BLK, dtype=jnp.int32)[None, :]
              ).astype(jnp.float32)

    a_full, bt, stop = pl.pallas_call(
        _proj_body,
        out_shape=(
            jax.ShapeDtypeStruct((N_NODES, 2 * N_BOND), jnp.float32),
            jax.ShapeDtypeStruct((2 * N_BOND, N_NODES), jnp.float32),
            jax.ShapeDtypeStruct((1, 1), jnp.float32),
        ),
    )(h_nodes, w1p, w2p, W_stop, biasa, b_stop.reshape(1, 1), sfeas)

    table = pl.pallas_call(
        _table_body,
        grid=(N_NODES // _I_BLK,),
        in_specs=[
            pl.BlockSpec((_I_BLK, N_NODES, EDGE_DIM), lambda i: (i, 0, 0)),
            pl.BlockSpec((EDGE_DIM, 2 * N_BOND), lambda i: (0, 0)),
            pl.BlockSpec((_I_BLK, 2 * N_BOND), lambda i: (i, 0)),
            pl.BlockSpec((2 * N_BOND, N_NODES), lambda i: (0, 0)),
            pl.BlockSpec((_BLK_P, _I_BLK), lambda i: (0, 0)),
        ],
        out_specs=pl.BlockSpec((_NQ, 2 * N_BOND, 128), lambda i: (i, 0, 0)),
        out_shape=jax.ShapeDtypeStruct((N_NODES * N_NODES // 128,
                                        2 * N_BOND, 128), jnp.float32),
    )(h_edges, w3p, a_full, bt, onehot)

    pad = _P_EDITS - N_EDITS
    ei1 = jnp.pad(edit_i, (0, pad)).astype(jnp.int32)
    ej1 = jnp.pad(edit_j, (0, pad)).astype(jnp.int32)
    eb1 = jnp.pad(edit_b, (0, pad)).astype(jnp.int32)
    ef1 = jnp.pad(feasible, (0, pad)).astype(jnp.int32)

    out1 = _make_sc_gather()(table.reshape(-1), ei1, ej1, eb1, ef1)

    return jnp.concatenate([out1[:N_EDITS], stop.reshape(1)])


# speedup vs baseline: 32.2129x; 1.0419x over previous
import jax
import jax.numpy as jnp
from jax import lax
from jax.experimental import pallas as pl
from jax.experimental.pallas import tpu as pltpu
from jax.experimental.pallas import tpu_sc as plsc

N_NODES = 512
NODE_DIM = 256
EDGE_DIM = 128
N_EDITS = 200000
N_BOND = 4

_NC = 2
_NS = 16
_NW = _NC * _NS
_EPW = 6272
_ROWS = _EPW // 128
_TAIL = N_EDITS - (_NW - 1) * _EPW
_TAIL_FULL = (_TAIL // 128) * 128
_TAIL_REM = _TAIL - _TAIL_FULL

_I_BLK = 32
_BLK_P = _I_BLK * N_NODES
_NQ = _BLK_P // 128


def _proj_body(h_ref, w1p_ref, w2p_ref, wstop_ref, biasa_ref, bstop_ref,
               sfeas_ref, a_ref, bt_ref, stop_ref):
    h = h_ref[...]
    a_ref[...] = jnp.dot(h, w1p_ref[...],
                         preferred_element_type=jnp.float32) + biasa_ref[...]
    bt_ref[...] = lax.dot_general(w2p_ref[...], h, (((0,), (1,)), ((), ())),
                                  preferred_element_type=jnp.float32)
    mean = jnp.mean(h, axis=0, keepdims=True)
    s = jnp.dot(mean, wstop_ref[...], preferred_element_type=jnp.float32)
    s = s + bstop_ref[...]
    stop_ref[...] = jnp.where(sfeas_ref[0, 0] > 0, s, -jnp.inf)


def _table_body(he_ref, w3p_ref, a_ref, bt_ref, oh_ref, t_ref):
    he = he_ref[...].reshape(_BLK_P, EDGE_DIM)
    e = lax.dot_general(w3p_ref[...], he, (((0,), (1,)), ((), ())),
                        preferred_element_type=jnp.float32)
    ea = lax.dot_general(a_ref[...], oh_ref[...], (((0,), (1,)), ((), ())),
                         preferred_element_type=jnp.float32)
    btile = jnp.concatenate([bt_ref[...]] * _I_BLK, axis=1)
    t = e + ea + btile
    for q in range(_NQ):
        t_ref[q] = t[:, q * 128:(q + 1) * 128]


def _sc_body(t_hbm, ei_hbm, ej_hbm, eb_hbm, ef_hbm, out_hbm,
             ei_v, ej_v, eb_v, ef_v, idx_v, val_v, sem):
    cid = lax.axis_index("c")
    sid = lax.axis_index("s")
    wid = sid * _NC + cid
    base = wid * _EPW
    last = _NW - 1
    nrows = jnp.where(wid == last, _TAIL // 128, _ROWS)

    @pl.when(wid < last)
    def _():
        pltpu.sync_copy(ei_hbm.at[pl.ds(base, _EPW)], ei_v)
        pltpu.sync_copy(ej_hbm.at[pl.ds(base, _EPW)], ej_v)
        pltpu.sync_copy(eb_hbm.at[pl.ds(base, _EPW)], eb_v)
        pltpu.sync_copy(ef_hbm.at[pl.ds(base, _EPW)], ef_v)

    @pl.when(wid == last)
    def _():
        tb = last * _EPW
        pltpu.sync_copy(ei_hbm.at[pl.ds(tb, _TAIL)], ei_v.at[pl.ds(0, _TAIL)])
        pltpu.sync_copy(ej_hbm.at[pl.ds(tb, _TAIL)], ej_v.at[pl.ds(0, _TAIL)])
        pltpu.sync_copy(eb_hbm.at[pl.ds(tb, _TAIL)], eb_v.at[pl.ds(0, _TAIL)])
        pltpu.sync_copy(ef_hbm.at[pl.ds(tb, _TAIL)], ef_v.at[pl.ds(0, _TAIL)])

    def idx_fire_row(r, carry):
        for c in range(8):
            sl = pl.ds(r * 128 + c * 16, 16)
            i16 = ei_v[sl]
            j16 = ej_v[sl]
            b16 = eb_v[sl]
            idx_v[sl] = ((i16 << 12) + ((j16 >> 7) << 10) + (b16 << 7)
                         + (j16 & 127))
        sl = pl.ds(r * 128, 128)
        pltpu.make_async_copy(t_hbm.at[idx_v.at[sl]], val_v.at[sl],
                              sem).start()
        return carry

    lax.fori_loop(0, nrows, idx_fire_row, 0, unroll=False)

    ninf = jnp.full((16,), -jnp.inf, dtype=jnp.float32)

    @pl.when(wid == last)
    def _():
        for c in range(_TAIL_REM // 16):
            sl = pl.ds(_TAIL_FULL + c * 16, 16)
            i16 = ei_v[sl]
            j16 = ej_v[sl]
            b16 = eb_v[sl]
            idx_v[sl] = ((i16 << 12) + ((j16 >> 7) << 10) + (b16 << 7)
                         + (j16 & 127))
        sl = pl.ds(_TAIL_FULL, _TAIL_REM)
        pltpu.make_async_copy(t_hbm.at[idx_v.at[sl]], val_v.at[sl],
                              sem).start()

    def drain_mask_row(r, carry):
        sl = pl.ds(r * 128, 128)
        pltpu.make_async_copy(t_hbm.at[idx_v.at[sl]], val_v.at[sl], sem).wait()
        for c in range(8):
            slc = pl.ds(r * 128 + c * 16, 16)
            val_v[slc] = jnp.where(ef_v[slc] > 0, val_v[slc], ninf)
        return carry

    lax.fori_loop(0, nrows, drain_mask_row, 0, unroll=False)

    @pl.when(wid == last)
    def _():
        sl = pl.ds(_TAIL_FULL, _TAIL_REM)
        pltpu.make_async_copy(t_hbm.at[idx_v.at[sl]], val_v.at[sl], sem).wait()
        for c in range(_TAIL_REM // 16):
            slc = pl.ds(_TAIL_FULL + c * 16, 16)
            val_v[slc] = jnp.where(ef_v[slc] > 0, val_v[slc], ninf)

    @pl.when(wid < last)
    def _():
        pltpu.sync_copy(val_v, out_hbm.at[pl.ds(base, _EPW)])

    @pl.when(wid == last)
    def _():
        pltpu.sync_copy(val_v.at[pl.ds(0, _TAIL)],
                        out_hbm.at[pl.ds(last * _EPW, _TAIL)])


def _make_sc_gather():
    return pl.kernel(
        _sc_body,
        out_type=jax.ShapeDtypeStruct((N_EDITS,), jnp.float32),
        mesh=plsc.VectorSubcoreMesh(core_axis_name="c", subcore_axis_name="s",
                                    num_cores=_NC, num_subcores=_NS),
        scratch_types=[
            pltpu.VMEM((_EPW,), jnp.int32),
            pltpu.VMEM((_EPW,), jnp.int32),
            pltpu.VMEM((_EPW,), jnp.int32),
            pltpu.VMEM((_EPW,), jnp.int32),
            pltpu.VMEM((_EPW,), jnp.int32),
            pltpu.VMEM((_EPW,), jnp.float32),
            pltpu.SemaphoreType.DMA,
        ],
    )


def kernel(h_nodes, h_edges, edit_i, edit_j, edit_b, feasible, stop_feasible,
           W_edit, b_edit, W_stop, b_stop):
    zpad = jnp.zeros((NODE_DIM, N_BOND), jnp.float32)
    w1p = jnp.concatenate([W_edit[:NODE_DIM], zpad], axis=1)
    w2p = jnp.concatenate([W_edit[NODE_DIM:2 * NODE_DIM], zpad], axis=1)
    w3p = jnp.pad(W_edit[2 * NODE_DIM:], ((0, 0), (0, 4)))
    biasa = jnp.concatenate([b_edit, jnp.zeros_like(b_edit)]).reshape(1, 8)
    sfeas = jnp.asarray(stop_feasible, jnp.int32).reshape(1, 1)
    onehot = (jnp.arange(_BLK_P, dtype=jnp.int32)[:, None] // N_NODES
              == jnp.arange(_I_BLK, dtype=jnp.int32)[None, :]
              ).astype(jnp.float32)

    a_full, bt, stop = pl.pallas_call(
        _proj_body,
        out_shape=(
            jax.ShapeDtypeStruct((N_NODES, 2 * N_BOND), jnp.float32),
            jax.ShapeDtypeStruct((2 * N_BOND, N_NODES), jnp.float32),
            jax.ShapeDtypeStruct((1, 1), jnp.float32),
        ),
    )(h_nodes, w1p, w2p, W_stop, biasa, b_stop.reshape(1, 1), sfeas)

    table = pl.pallas_call(
        _table_body,
        grid=(N_NODES // _I_BLK,),
        in_specs=[
            pl.BlockSpec((_I_BLK, N_NODES, EDGE_DIM), lambda i: (i, 0, 0)),
            pl.BlockSpec((EDGE_DIM, 2 * N_BOND), lambda i: (0, 0)),
            pl.BlockSpec((_I_BLK, 2 * N_BOND), lambda i: (i, 0)),
            pl.BlockSpec((2 * N_BOND, N_NODES), lambda i: (0, 0)),
            pl.BlockSpec((_BLK_P, _I_BLK), lambda i: (0, 0)),
        ],
        out_specs=pl.BlockSpec((_NQ, 2 * N_BOND, 128), lambda i: (i, 0, 0)),
        out_shape=jax.ShapeDtypeStruct((N_NODES * N_NODES // 128,
                                        2 * N_BOND, 128), jnp.float32),
    )(h_edges, w3p, a_full, bt, onehot)

    out1 = _make_sc_gather()(table.reshape(-1),
                             edit_i.astype(jnp.int32),
                             edit_j.astype(jnp.int32),
                             edit_b.astype(jnp.int32),
                             feasible.astype(jnp.int32))

    return jnp.concatenate([out1, stop.reshape(1)])
